# initial kernel scaffold (unmeasured)
import jax
import jax.numpy as jnp
from jax import lax
from jax.experimental import pallas as pl
from jax.experimental.pallas import tpu as pltpu


def kernel(
    x,
):
    def body(*refs):
        pass

    out_shape = jax.ShapeDtypeStruct(..., jnp.float32)
    return pl.pallas_call(body, out_shape=out_shape)(...)


# baseline (device time: 249355 ns/iter reference)
import jax
import jax.numpy as jnp
from jax import lax
from jax.experimental import pallas as pl
from jax.experimental.pallas import tpu as pltpu

M = 8192
N_FULL = 2048
N_OUT = 1024
M_HALF = 4096


def kernel(x):
    xb = x[0].astype(jnp.bfloat16)

    def body(x_ref, out_ref, recv_ref, sem_s1, sem_r1, sem_s2, sem_r2):
        my_x = lax.axis_index("x")
        my_y = lax.axis_index("y")
        y_peer = (my_x, 1 - my_y)
        x_peer = (1 - my_x, my_y)

        barrier = pltpu.get_barrier_semaphore()
        for nbr in (y_peer, x_peer):
            pl.semaphore_signal(
                barrier, inc=1, device_id=nbr,
                device_id_type=pl.DeviceIdType.MESH,
            )
        pl.semaphore_wait(barrier, 2)

        rows = pl.ds(my_x * M_HALF, M_HALF)
        my_col = pl.ds(my_y * N_OUT, N_OUT)
        peer_col = pl.ds((1 - my_y) * N_OUT, N_OUT)

        rdma1 = pltpu.make_async_remote_copy(
            src_ref=x_ref.at[rows, peer_col],
            dst_ref=recv_ref,
            send_sem=sem_s1,
            recv_sem=sem_r1,
            device_id=y_peer,
            device_id_type=pl.DeviceIdType.MESH,
        )
        rdma1.start()
        rdma1.wait()

        out_ref[rows, :] = x_ref[rows, my_col] + recv_ref[...]

        rdma2 = pltpu.make_async_remote_copy(
            src_ref=out_ref.at[rows, :],
            dst_ref=out_ref.at[rows, :],
            send_sem=sem_s2,
            recv_sem=sem_r2,
            device_id=x_peer,
            device_id_type=pl.DeviceIdType.MESH,
        )
        rdma2.start()
        rdma2.wait()

    return pl.pallas_call(
        body,
        out_shape=jax.ShapeDtypeStruct((M, N_OUT), jnp.bfloat16),
        in_specs=[pl.BlockSpec(memory_space=pltpu.VMEM)],
        out_specs=pl.BlockSpec(memory_space=pltpu.VMEM),
        scratch_shapes=[
            pltpu.VMEM((M_HALF, N_OUT), jnp.bfloat16),
            pltpu.SemaphoreType.DMA,
            pltpu.SemaphoreType.DMA,
            pltpu.SemaphoreType.DMA,
            pltpu.SemaphoreType.DMA,
        ],
        compiler_params=pltpu.CompilerParams(
            collective_id=0,
            vmem_limit_bytes=100 * 1024 * 1024,
        ),
    )(xb)


# device time: 129382 ns/iter; 1.9273x vs baseline; 1.9273x over previous
import jax
import jax.numpy as jnp
from jax import lax
from jax.experimental import pallas as pl
from jax.experimental.pallas import tpu as pltpu

M = 8192
N_FULL = 2048
N_OUT = 1024
M_HALF = 4096
C = 16
CM = M_HALF // C


def kernel(x):
    def body(x_hbm, out_ref, stage, send1, loc, recv1,
             copy_sems, s1, r1, s2, r2):
        my_x = lax.axis_index("x")
        my_y = lax.axis_index("y")
        y_peer = (my_x, 1 - my_y)
        x_peer = (1 - my_x, my_y)

        barrier = pltpu.get_barrier_semaphore()
        for nbr in (y_peer, x_peer):
            pl.semaphore_signal(
                barrier, inc=1, device_id=nbr,
                device_id_type=pl.DeviceIdType.MESH,
            )
        pl.semaphore_wait(barrier, 2)

        row0 = my_x * M_HALF
        my_col = pl.ds(my_y * N_OUT, N_OUT)
        peer_col = pl.ds((1 - my_y) * N_OUT, N_OUT)

        def stage_copy(i):
            return pltpu.make_async_copy(
                x_hbm.at[0, pl.ds(row0 + i * CM, CM), :],
                stage.at[i % 2],
                copy_sems.at[i % 2],
            )

        def rdma1(i):
            return pltpu.make_async_remote_copy(
                src_ref=send1.at[pl.ds(i * CM, CM), :],
                dst_ref=recv1.at[pl.ds(i * CM, CM), :],
                send_sem=s1.at[i],
                recv_sem=r1.at[i],
                device_id=y_peer,
                device_id_type=pl.DeviceIdType.MESH,
            )

        def rdma2(i):
            chunk = pl.ds(row0 + i * CM, CM)
            return pltpu.make_async_remote_copy(
                src_ref=out_ref.at[chunk, :],
                dst_ref=out_ref.at[chunk, :],
                send_sem=s2.at[i],
                recv_sem=r2.at[i],
                device_id=x_peer,
                device_id_type=pl.DeviceIdType.MESH,
            )

        stage_copy(0).start()
        for i in range(C):
            if i + 1 < C:
                stage_copy(i + 1).start()
            stage_copy(i).wait()
            rows_i = pl.ds(i * CM, CM)
            send1[rows_i, :] = stage[i % 2, :, peer_col].astype(jnp.bfloat16)
            loc[rows_i, :] = stage[i % 2, :, my_col].astype(jnp.bfloat16)
            rdma1(i).start()

        for i in range(C):
            rdma1(i).wait_recv()
            rows_i = pl.ds(i * CM, CM)
            out_ref[pl.ds(row0 + i * CM, CM), :] = loc[rows_i, :] + recv1[rows_i, :]
            rdma2(i).start()

        for i in range(C):
            rdma2(i).wait_recv()
        for i in range(C):
            rdma1(i).wait_send()
            rdma2(i).wait_send()

    return pl.pallas_call(
        body,
        out_shape=jax.ShapeDtypeStruct((M, N_OUT), jnp.bfloat16),
        in_specs=[pl.BlockSpec(memory_space=pl.ANY)],
        out_specs=pl.BlockSpec(memory_space=pltpu.VMEM),
        scratch_shapes=[
            pltpu.VMEM((2, CM, N_FULL), jnp.float32),
            pltpu.VMEM((M_HALF, N_OUT), jnp.bfloat16),
            pltpu.VMEM((M_HALF, N_OUT), jnp.bfloat16),
            pltpu.VMEM((M_HALF, N_OUT), jnp.bfloat16),
            pltpu.SemaphoreType.DMA((2,)),
            pltpu.SemaphoreType.DMA((C,)),
            pltpu.SemaphoreType.DMA((C,)),
            pltpu.SemaphoreType.DMA((C,)),
            pltpu.SemaphoreType.DMA((C,)),
        ],
        compiler_params=pltpu.CompilerParams(
            collective_id=0,
            vmem_limit_bytes=100 * 1024 * 1024,
        ),
    )(x)


# device time: 124066 ns/iter; 2.0099x vs baseline; 1.0428x over previous
import jax
import jax.numpy as jnp
from jax import lax
from jax.experimental import pallas as pl
from jax.experimental.pallas import tpu as pltpu

M = 8192
N_FULL = 2048
N_OUT = 1024
M_HALF = 4096
C = 16
CM = M_HALF // C


def kernel(x):
    def body(x_hbm, out_hbm, stage, send1, loc, recv1, red,
             copy_sems, out_sems, s1, r1, s2, r2):
        my_x = lax.axis_index("x")
        my_y = lax.axis_index("y")
        y_peer = (my_x, 1 - my_y)
        x_peer = (1 - my_x, my_y)

        barrier = pltpu.get_barrier_semaphore()
        for nbr in (y_peer, x_peer):
            pl.semaphore_signal(
                barrier, inc=1, device_id=nbr,
                device_id_type=pl.DeviceIdType.MESH,
            )
        pl.semaphore_wait(barrier, 2)

        row0 = my_x * M_HALF
        my_col = pl.ds(my_y * N_OUT, N_OUT)
        peer_col = pl.ds((1 - my_y) * N_OUT, N_OUT)

        def stage_copy(i):
            return pltpu.make_async_copy(
                x_hbm.at[0, pl.ds(row0 + i * CM, CM), :],
                stage.at[i % 2],
                copy_sems.at[i % 2],
            )

        def rdma1(i):
            return pltpu.make_async_remote_copy(
                src_ref=send1.at[pl.ds(i * CM, CM), :],
                dst_ref=recv1.at[pl.ds(i * CM, CM), :],
                send_sem=s1.at[i],
                recv_sem=r1.at[i],
                device_id=y_peer,
                device_id_type=pl.DeviceIdType.MESH,
            )

        def rdma2(i):
            return pltpu.make_async_remote_copy(
                src_ref=red.at[pl.ds(i * CM, CM), :],
                dst_ref=out_hbm.at[pl.ds(row0 + i * CM, CM), :],
                send_sem=s2.at[i],
                recv_sem=r2.at[i],
                device_id=x_peer,
                device_id_type=pl.DeviceIdType.MESH,
            )

        stage_copy(0).start()
        for i in range(C):
            if i + 1 < C:
                stage_copy(i + 1).start()
            stage_copy(i).wait()
            rows_i = pl.ds(i * CM, CM)
            send1[rows_i, :] = stage[i % 2, :, peer_col].astype(jnp.bfloat16)
            loc[rows_i, :] = stage[i % 2, :, my_col].astype(jnp.bfloat16)
            rdma1(i).start()

        for i in range(C):
            rdma1(i).wait_recv()
            rows_i = pl.ds(i * CM, CM)
            red[rows_i, :] = loc[rows_i, :] + recv1[rows_i, :]
            rdma2(i).start()
            pltpu.make_async_copy(
                red.at[rows_i, :],
                out_hbm.at[pl.ds(row0 + i * CM, CM), :],
                out_sems.at[i],
            ).start()

        for i in range(C):
            rdma2(i).wait_recv()
            pltpu.make_async_copy(
                red.at[pl.ds(i * CM, CM), :],
                out_hbm.at[pl.ds(row0 + i * CM, CM), :],
                out_sems.at[i],
            ).wait()
        for i in range(C):
            rdma1(i).wait_send()
            rdma2(i).wait_send()

    return pl.pallas_call(
        body,
        out_shape=jax.ShapeDtypeStruct((M, N_OUT), jnp.bfloat16),
        in_specs=[pl.BlockSpec(memory_space=pl.ANY)],
        out_specs=pl.BlockSpec(memory_space=pl.ANY),
        scratch_shapes=[
            pltpu.VMEM((2, CM, N_FULL), jnp.float32),
            pltpu.VMEM((M_HALF, N_OUT), jnp.bfloat16),
            pltpu.VMEM((M_HALF, N_OUT), jnp.bfloat16),
            pltpu.VMEM((M_HALF, N_OUT), jnp.bfloat16),
            pltpu.VMEM((M_HALF, N_OUT), jnp.bfloat16),
            pltpu.SemaphoreType.DMA((2,)),
            pltpu.SemaphoreType.DMA((C,)),
            pltpu.SemaphoreType.DMA((C,)),
            pltpu.SemaphoreType.DMA((C,)),
            pltpu.SemaphoreType.DMA((C,)),
            pltpu.SemaphoreType.DMA((C,)),
        ],
        compiler_params=pltpu.CompilerParams(
            collective_id=0,
            vmem_limit_bytes=100 * 1024 * 1024,
        ),
    )(x)


# device time: 123732 ns/iter; 2.0153x vs baseline; 1.0027x over previous
import jax
import jax.numpy as jnp
from jax import lax
from jax.experimental import pallas as pl
from jax.experimental.pallas import tpu as pltpu

M = 8192
N_FULL = 2048
N_OUT = 1024
M_HALF = 4096
C = 8
CM = M_HALF // C


def kernel(x):
    def body(x_hbm, out_hbm, stage, send1, loc, recv1, red,
             copy_sems, out_sems, s1, r1, s2, r2):
        my_x = lax.axis_index("x")
        my_y = lax.axis_index("y")
        y_peer = (my_x, 1 - my_y)
        x_peer = (1 - my_x, my_y)

        barrier = pltpu.get_barrier_semaphore()
        for nbr in (y_peer, x_peer):
            pl.semaphore_signal(
                barrier, inc=1, device_id=nbr,
                device_id_type=pl.DeviceIdType.MESH,
            )
        pl.semaphore_wait(barrier, 2)

        row0 = my_x * M_HALF
        my_col = pl.ds(my_y * N_OUT, N_OUT)
        peer_col = pl.ds((1 - my_y) * N_OUT, N_OUT)

        def stage_copy(i):
            return pltpu.make_async_copy(
                x_hbm.at[0, pl.ds(row0 + i * CM, CM), :],
                stage.at[i % 2],
                copy_sems.at[i % 2],
            )

        def rdma1(i):
            return pltpu.make_async_remote_copy(
                src_ref=send1.at[pl.ds(i * CM, CM), :],
                dst_ref=recv1.at[pl.ds(i * CM, CM), :],
                send_sem=s1.at[i],
                recv_sem=r1.at[i],
                device_id=y_peer,
                device_id_type=pl.DeviceIdType.MESH,
            )

        def rdma2(i):
            return pltpu.make_async_remote_copy(
                src_ref=red.at[pl.ds(i * CM, CM), :],
                dst_ref=out_hbm.at[pl.ds(row0 + i * CM, CM), :],
                send_sem=s2.at[i],
                recv_sem=r2.at[i],
                device_id=x_peer,
                device_id_type=pl.DeviceIdType.MESH,
            )

        stage_copy(0).start()
        for i in range(C):
            if i + 1 < C:
                stage_copy(i + 1).start()
            stage_copy(i).wait()
            rows_i = pl.ds(i * CM, CM)
            send1[rows_i, :] = stage[i % 2, :, peer_col].astype(jnp.bfloat16)
            loc[rows_i, :] = stage[i % 2, :, my_col].astype(jnp.bfloat16)
            rdma1(i).start()

        for i in range(C):
            rdma1(i).wait_recv()
            rows_i = pl.ds(i * CM, CM)
            red[rows_i, :] = loc[rows_i, :] + recv1[rows_i, :]
            rdma2(i).start()
            pltpu.make_async_copy(
                red.at[rows_i, :],
                out_hbm.at[pl.ds(row0 + i * CM, CM), :],
                out_sems.at[i],
            ).start()

        for i in range(C):
            rdma2(i).wait_recv()
            pltpu.make_async_copy(
                red.at[pl.ds(i * CM, CM), :],
                out_hbm.at[pl.ds(row0 + i * CM, CM), :],
                out_sems.at[i],
            ).wait()
        for i in range(C):
            rdma1(i).wait_send()
            rdma2(i).wait_send()

    return pl.pallas_call(
        body,
        out_shape=jax.ShapeDtypeStruct((M, N_OUT), jnp.bfloat16),
        in_specs=[pl.BlockSpec(memory_space=pl.ANY)],
        out_specs=pl.BlockSpec(memory_space=pl.ANY),
        scratch_shapes=[
            pltpu.VMEM((2, CM, N_FULL), jnp.float32),
            pltpu.VMEM((M_HALF, N_OUT), jnp.bfloat16),
            pltpu.VMEM((M_HALF, N_OUT), jnp.bfloat16),
            pltpu.VMEM((M_HALF, N_OUT), jnp.bfloat16),
            pltpu.VMEM((M_HALF, N_OUT), jnp.bfloat16),
            pltpu.SemaphoreType.DMA((2,)),
            pltpu.SemaphoreType.DMA((C,)),
            pltpu.SemaphoreType.DMA((C,)),
            pltpu.SemaphoreType.DMA((C,)),
            pltpu.SemaphoreType.DMA((C,)),
            pltpu.SemaphoreType.DMA((C,)),
        ],
        compiler_params=pltpu.CompilerParams(
            collective_id=0,
            vmem_limit_bytes=100 * 1024 * 1024,
        ),
    )(x)


# device time: 123646 ns/iter; 2.0167x vs baseline; 1.0007x over previous
import jax
import jax.numpy as jnp
from jax import lax
from jax.experimental import pallas as pl
from jax.experimental.pallas import tpu as pltpu

M = 8192
N_FULL = 2048
N_OUT = 1024
M_HALF = 4096

CHUNKS = [128, 128] + [256] * 14 + [128, 128]
assert sum(CHUNKS) == M_HALF
OFFS = [sum(CHUNKS[:i]) for i in range(len(CHUNKS))]
C = len(CHUNKS)
CM_MAX = max(CHUNKS)


def kernel(x):
    def body(x_hbm, out_hbm, stage, send1, loc, recv1, red,
             copy_sems, out_sems, s1, r1, s2, r2):
        my_x = lax.axis_index("x")
        my_y = lax.axis_index("y")
        y_peer = (my_x, 1 - my_y)
        x_peer = (1 - my_x, my_y)

        row0 = my_x * M_HALF
        my_col = pl.ds(my_y * N_OUT, N_OUT)
        peer_col = pl.ds((1 - my_y) * N_OUT, N_OUT)

        def stage_copy(i):
            return pltpu.make_async_copy(
                x_hbm.at[0, pl.ds(row0 + OFFS[i], CHUNKS[i]), :],
                stage.at[i % 2, pl.ds(0, CHUNKS[i])],
                copy_sems.at[i % 2],
            )

        def rdma1(i):
            return pltpu.make_async_remote_copy(
                src_ref=send1.at[pl.ds(OFFS[i], CHUNKS[i]), :],
                dst_ref=recv1.at[pl.ds(OFFS[i], CHUNKS[i]), :],
                send_sem=s1.at[i],
                recv_sem=r1.at[i],
                device_id=y_peer,
                device_id_type=pl.DeviceIdType.MESH,
            )

        def rdma2(i):
            return pltpu.make_async_remote_copy(
                src_ref=red.at[pl.ds(OFFS[i], CHUNKS[i]), :],
                dst_ref=out_hbm.at[pl.ds(row0 + OFFS[i], CHUNKS[i]), :],
                send_sem=s2.at[i],
                recv_sem=r2.at[i],
                device_id=x_peer,
                device_id_type=pl.DeviceIdType.MESH,
            )

        stage_copy(0).start()
        stage_copy(1).start()

        barrier = pltpu.get_barrier_semaphore()
        for nbr in (y_peer, x_peer):
            pl.semaphore_signal(
                barrier, inc=1, device_id=nbr,
                device_id_type=pl.DeviceIdType.MESH,
            )
        pl.semaphore_wait(barrier, 2)

        for i in range(C):
            stage_copy(i).wait()
            rows_i = pl.ds(OFFS[i], CHUNKS[i])
            srows = pl.ds(0, CHUNKS[i])
            send1[rows_i, :] = stage[i % 2, srows, peer_col].astype(jnp.bfloat16)
            loc[rows_i, :] = stage[i % 2, srows, my_col].astype(jnp.bfloat16)
            rdma1(i).start()
            if i + 2 < C:
                stage_copy(i + 2).start()

        for i in range(C):
            rdma1(i).wait_recv()
            rows_i = pl.ds(OFFS[i], CHUNKS[i])
            red[rows_i, :] = loc[rows_i, :] + recv1[rows_i, :]
            rdma2(i).start()
            pltpu.make_async_copy(
                red.at[rows_i, :],
                out_hbm.at[pl.ds(row0 + OFFS[i], CHUNKS[i]), :],
                out_sems.at[i],
            ).start()

        for i in range(C):
            rdma2(i).wait_recv()
            pltpu.make_async_copy(
                red.at[pl.ds(OFFS[i], CHUNKS[i]), :],
                out_hbm.at[pl.ds(row0 + OFFS[i], CHUNKS[i]), :],
                out_sems.at[i],
            ).wait()
        for i in range(C):
            rdma1(i).wait_send()
            rdma2(i).wait_send()

    return pl.pallas_call(
        body,
        out_shape=jax.ShapeDtypeStruct((M, N_OUT), jnp.bfloat16),
        in_specs=[pl.BlockSpec(memory_space=pl.ANY)],
        out_specs=pl.BlockSpec(memory_space=pl.ANY),
        scratch_shapes=[
            pltpu.VMEM((2, CM_MAX, N_FULL), jnp.float32),
            pltpu.VMEM((M_HALF, N_OUT), jnp.bfloat16),
            pltpu.VMEM((M_HALF, N_OUT), jnp.bfloat16),
            pltpu.VMEM((M_HALF, N_OUT), jnp.bfloat16),
            pltpu.VMEM((M_HALF, N_OUT), jnp.bfloat16),
            pltpu.SemaphoreType.DMA((2,)),
            pltpu.SemaphoreType.DMA((C,)),
            pltpu.SemaphoreType.DMA((C,)),
            pltpu.SemaphoreType.DMA((C,)),
            pltpu.SemaphoreType.DMA((C,)),
            pltpu.SemaphoreType.DMA((C,)),
        ],
        compiler_params=pltpu.CompilerParams(
            collective_id=0,
            vmem_limit_bytes=100 * 1024 * 1024,
        ),
    )(x)
